# baseline (speedup 1.0000x reference)
import functools

import jax
import jax.numpy as jnp
from jax import lax
from jax.experimental import pallas as pl
from jax.experimental.pallas import tpu as pltpu, tpu_sc as plsc

N = 50000
E = 200000
R = 3
D_IN, D_H, D_OUT = 128, 128, 64

N_PAD = 50176
E_PAD = 204800
CH = 320
CHUNKS = E_PAD // (16 * CH)
ROWS_T = N_PAD // 16
B_ROWS = 512
FC = 32


def _mm_body(h_ref, w_ref, o_ref):
    o_ref[...] = jnp.dot(h_ref[...], w_ref[...],
                         preferred_element_type=jnp.float32)


def _mm(h, w):
    n = h.shape[0]
    return pl.pallas_call(
        _mm_body,
        grid=(n // B_ROWS,),
        in_specs=[
            pl.BlockSpec((B_ROWS, h.shape[1]), lambda i: (i, 0)),
            pl.BlockSpec(w.shape, lambda i: (0, 0)),
        ],
        out_specs=pl.BlockSpec((B_ROWS, w.shape[1]), lambda i: (i, 0)),
        out_shape=jax.ShapeDtypeStruct((n, w.shape[1]), jnp.float32),
    )(h, w)


def _epi_body(agg_ref, deg_ref, h_ref, wl_ref, b_ref, wn_ref,
              hout_ref, yout_ref, *, relu):
    deg = deg_ref[...]
    nrm = 1.0 / jnp.maximum(deg, 1.0)
    agg = agg_ref[...]
    s = jnp.sum(agg * nrm, axis=0)
    h = s + jnp.dot(h_ref[...], wl_ref[...],
                    preferred_element_type=jnp.float32) + b_ref[...]
    if relu:
        h = jnp.maximum(h, 0.0)
    hout_ref[...] = h
    if yout_ref is not None:
        yout_ref[...] = jnp.dot(h, wn_ref[...],
                                preferred_element_type=jnp.float32)


def _epi(agg, deg, h, wl, b, wn, relu):
    d = agg.shape[2]
    have_y = wn is not None
    body = functools.partial(_epi_body, relu=relu)
    if not have_y:
        def body(agg_ref, deg_ref, h_ref, wl_ref, b_ref, hout_ref):
            _epi_body(agg_ref, deg_ref, h_ref, wl_ref, b_ref, None,
                      hout_ref, None, relu=relu)
    in_specs = [
        pl.BlockSpec((R, B_ROWS, d), lambda i: (0, i, 0)),
        pl.BlockSpec((R, B_ROWS, 1), lambda i: (0, i, 0)),
        pl.BlockSpec((B_ROWS, h.shape[1]), lambda i: (i, 0)),
        pl.BlockSpec(wl.shape, lambda i: (0, 0)),
        pl.BlockSpec(b.shape, lambda i: (0, 0)),
    ]
    out_shape = [jax.ShapeDtypeStruct((N_PAD, d), jnp.float32)]
    out_specs = [pl.BlockSpec((B_ROWS, d), lambda i: (i, 0))]
    args = [agg, deg, h, wl, b]
    if have_y:
        in_specs.append(pl.BlockSpec(wn.shape, lambda i: (0, 0)))
        out_shape.append(
            jax.ShapeDtypeStruct((N_PAD, wn.shape[1]), jnp.float32))
        out_specs.append(pl.BlockSpec((B_ROWS, wn.shape[1]), lambda i: (i, 0)))
        args.append(wn)
    res = pl.pallas_call(
        body,
        grid=(N_PAD // B_ROWS,),
        in_specs=in_specs,
        out_specs=out_specs,
        out_shape=out_shape,
    )(*args)
    return res if have_y else (res[0], None)


_MESH = plsc.VectorSubcoreMesh(core_axis_name="c", subcore_axis_name="s",
                               num_cores=2, num_subcores=16)
_SC_PARAMS = pltpu.CompilerParams(use_tc_tiling_on_sc=False)


def _fill(ref, n, value):
    def body(i, carry):
        ref[pl.ds(i * 16, 16)] = jnp.full((16,), value, jnp.float32)
        return carry
    lax.fori_loop(0, n // 16, body, 0)


def _deg_kernel(dst3_hbm, deg_hbm, ones_v, idx_v, zbuf, acc):
    c = lax.axis_index("c")
    t = lax.axis_index("s")
    _fill(ones_v, CH, 1.0)
    z = 3 * N_PAD // 16
    _fill(zbuf, z, 0.0)
    pltpu.sync_copy(zbuf, acc.at[pl.ds(t * z, z)])
    plsc.subcore_barrier()

    def run_rel(r):
        def body(i, carry):
            off = r * E_PAD + t * (CHUNKS * CH) + i * CH
            pltpu.sync_copy(dst3_hbm.at[pl.ds(off, CH)], idx_v)
            pltpu.sync_copy(ones_v, acc.at[idx_v], add=True)
            return carry
        lax.fori_loop(0, CHUNKS, body, 0)

    @pl.when(c == 0)
    def _():
        run_rel(0)
        run_rel(1)

    @pl.when(c == 1)
    def _():
        run_rel(2)

    plsc.subcore_barrier()

    @pl.when(c == 0)
    def _():
        sl = 2 * N_PAD // 16
        pltpu.sync_copy(acc.at[pl.ds(t * sl, sl)], zbuf.at[pl.ds(0, sl)])
        pltpu.sync_copy(zbuf.at[pl.ds(0, sl)], deg_hbm.at[pl.ds(t * sl, sl)])

    @pl.when(c == 1)
    def _():
        sl = N_PAD // 16
        pltpu.sync_copy(acc.at[pl.ds(2 * N_PAD + t * sl, sl)],
                        zbuf.at[pl.ds(0, sl)])
        pltpu.sync_copy(zbuf.at[pl.ds(0, sl)],
                        deg_hbm.at[pl.ds(2 * N_PAD + t * sl, sl)])


@functools.partial(
    pl.kernel,
    out_type=jax.ShapeDtypeStruct((3 * N_PAD,), jnp.float32),
    mesh=_MESH,
    scratch_types=[
        pltpu.VMEM((CH,), jnp.float32),
        pltpu.VMEM((CH,), jnp.int32),
        pltpu.VMEM((3 * N_PAD // 16,), jnp.float32),
        pltpu.VMEM_SHARED((3 * N_PAD,), jnp.float32),
    ],
    compiler_params=_SC_PARAMS,
)
def _deg(dst3_hbm, deg_hbm, ones_v, idx_v, zbuf, acc):
    _deg_kernel(dst3_hbm, deg_hbm, ones_v, idx_v, zbuf, acc)


def _fill2d(ref, nrows, value):
    def body(i, carry):
        for j in range(FC // 16):
            ref[i, pl.ds(j * 16, 16)] = jnp.full((16,), value, jnp.float32)
        return carry
    lax.fori_loop(0, nrows, body, 0)


def _make_agg(dout):
    nchunk = dout // FC
    passes = [(r, p) for r in range(R) for p in range(nchunk)]
    half = len(passes) // 2
    zrows = ROWS_T // 14
    nz = 14

    def body(y_hbm, src_hbm, dst_hbm, agg_hbm, acc, sem0, sem1):
        pl.run_scoped(
            functools.partial(_agg_scoped, y_hbm, src_hbm, dst_hbm,
                              agg_hbm, acc, sem0, sem1),
            sb=[pltpu.VMEM((CH,), jnp.int32) for _ in range(4)],
            db=[pltpu.VMEM((CH,), jnp.int32) for _ in range(2)],
            rows0=pltpu.VMEM((CH, FC), jnp.float32),
            rows1=pltpu.VMEM((CH, FC), jnp.float32),
            zstage=pltpu.VMEM((zrows, FC), jnp.float32),
        )

    def _agg_scoped(y_hbm, src_hbm, dst_hbm, agg_hbm, acc, sem0, sem1,
                    sb, db, rows0, rows1, zstage):
        c = lax.axis_index("c")
        t = lax.axis_index("s")
        half = (R * nchunk) // 2
        quads = CHUNKS // 4

        def pass_body(q, carry):
            r = q // nchunk
            p = q - r * nchunk
            sbase = q * (E_PAD // CH) + t * CHUNKS
            dbase = r * (E_PAD // CH) + t * CHUNKS

            _fill2d(zstage, zrows, 0.0)

            def zero(zz, carry2):
                pltpu.sync_copy(zstage,
                                acc.at[pl.ds(t * ROWS_T + zz * zrows, zrows)])
                return carry2
            lax.fori_loop(0, nz, zero, 0)
            plsc.subcore_barrier()

            for k in range(4):
                pltpu.sync_copy(src_hbm.at[sbase + k], sb[k])

            def quad(h, carry2):
                ch = 4 * h
                cp0 = pltpu.async_copy(y_hbm.at[sb[0]], rows0, sem0)
                cp1 = pltpu.async_copy(y_hbm.at[sb[1]], rows1, sem1)
                pltpu.sync_copy(dst_hbm.at[dbase + ch], db[0])
                pltpu.sync_copy(dst_hbm.at[dbase + ch + 1], db[1])
                cp0.wait()
                pltpu.sync_copy(rows0, acc.at[db[0]], add=True)
                cp2 = pltpu.async_copy(y_hbm.at[sb[2]], rows0, sem0)
                pltpu.sync_copy(src_hbm.at[sbase + ch + 4], sb[0])
                pltpu.sync_copy(dst_hbm.at[dbase + ch + 2], db[0])
                cp1.wait()
                pltpu.sync_copy(rows1, acc.at[db[1]], add=True)
                cp3 = pltpu.async_copy(y_hbm.at[sb[3]], rows1, sem1)
                pltpu.sync_copy(src_hbm.at[sbase + ch + 5], sb[1])
                pltpu.sync_copy(dst_hbm.at[dbase + ch + 3], db[1])
                cp2.wait()
                pltpu.sync_copy(rows0, acc.at[db[0]], add=True)
                pltpu.sync_copy(src_hbm.at[sbase + ch + 6], sb[2])
                cp3.wait()
                pltpu.sync_copy(rows1, acc.at[db[1]], add=True)
                pltpu.sync_copy(src_hbm.at[sbase + ch + 7], sb[3])
                return carry2
            lax.fori_loop(0, quads, quad, 0)
            plsc.subcore_barrier()

            def drain(zz, carry2):
                pltpu.sync_copy(
                    acc.at[pl.ds(t * ROWS_T + zz * zrows, zrows)], zstage)
                pltpu.sync_copy(
                    zstage,
                    agg_hbm.at[
                        pl.ds(r * N_PAD + t * ROWS_T + zz * zrows, zrows),
                        pl.ds(p * FC, FC)])
                return carry2
            lax.fori_loop(0, nz, drain, 0)
            return carry

        lax.fori_loop(c * half, (c + 1) * half, pass_body, 0)

    return pl.kernel(
        body,
        out_type=jax.ShapeDtypeStruct((R * N_PAD, dout), jnp.float32),
        mesh=_MESH,
        scratch_types=[
            pltpu.VMEM_SHARED((N_PAD, FC), jnp.float32),
            pltpu.SemaphoreType.DMA,
            pltpu.SemaphoreType.DMA,
        ],
        compiler_params=_SC_PARAMS,
    )


_agg128 = _make_agg(D_H)
_agg64 = _make_agg(D_OUT)


def kernel(x, edge_index_r0, edge_index_r1, edge_index_r2,
           W0_r0, W0_r1, W0_r2, W0_loop, b0_loop,
           W1_r0, W1_r1, W1_r2, W1_loop, b1_loop,
           W2_r0, W2_r1, W2_r2, W2_loop, b2_loop):
    pad_n = E_PAD - E
    pad_src = (jnp.arange(pad_n, dtype=jnp.int32) * 977) % N
    pad_dst = N + (jnp.arange(pad_n, dtype=jnp.int32) % (N_PAD - N))
    srcs, dsts, dst3s = [], [], []
    for r, ei in enumerate((edge_index_r0, edge_index_r1, edge_index_r2)):
        s = jnp.concatenate([ei[0], pad_src])
        d = jnp.concatenate([ei[1], pad_dst])
        srcs.append(s)
        dsts.append(d)
        dst3s.append(d + r * N_PAD)
    dst_flat = jnp.concatenate(dsts)
    dst3_flat = jnp.concatenate(dst3s)
    srcs_pad = jnp.stack(srcs)
    q12 = jnp.arange(12, dtype=jnp.int32).reshape(3, 4)
    src12 = (srcs_pad[:, None, :] * 12 + q12[:, :, None]).reshape(-1)
    q6 = jnp.arange(6, dtype=jnp.int32).reshape(3, 2)
    src6 = (srcs_pad[:, None, :] * 6 + q6[:, :, None]).reshape(-1)

    x_pad = jnp.pad(x, ((0, N_PAD - N), (0, 0)))
    w_all0 = jnp.concatenate([W0_r0, W0_r1, W0_r2], axis=1)
    w_all1 = jnp.concatenate([W1_r0, W1_r1, W1_r2], axis=1)
    w_all2 = jnp.concatenate([W2_r0, W2_r1, W2_r2], axis=1)
    b0 = b0_loop.reshape(1, -1)
    b1 = b1_loop.reshape(1, -1)
    b2 = b2_loop.reshape(1, -1)

    deg = _deg(dst3_flat).reshape(R, N_PAD, 1)

    y0 = _mm(x_pad, w_all0)
    src12_2d = jnp.pad(src12.reshape(-1, CH), ((0, 8), (0, 0)))
    src6_2d = jnp.pad(src6.reshape(-1, CH), ((0, 8), (0, 0)))
    dst_2d = dst_flat.reshape(-1, CH)
    agg0 = _agg128(y0.reshape(-1, FC), src12_2d, dst_2d).reshape(R, N_PAD, D_H)
    h1, y1 = _epi(agg0, deg, x_pad, W0_loop, b0, w_all1, relu=True)

    agg1 = _agg128(y1.reshape(-1, FC), src12_2d, dst_2d).reshape(R, N_PAD, D_H)
    h2, y2 = _epi(agg1, deg, h1, W1_loop, b1, w_all2, relu=True)

    agg2 = _agg64(y2.reshape(-1, FC), src6_2d, dst_2d).reshape(R, N_PAD, D_OUT)
    h3, _ = _epi(agg2, deg, h2, W2_loop, b2, None, relu=False)

    return h3[:N]


# speedup vs baseline: 1.2154x; 1.2154x over previous
import functools

import jax
import jax.numpy as jnp
from jax import lax
from jax.experimental import pallas as pl
from jax.experimental.pallas import tpu as pltpu, tpu_sc as plsc

N = 50000
E = 200000
R = 3
D_IN, D_H, D_OUT = 128, 128, 64

N_PAD = 50176
E_PAD = 204800
CH = 400
CHUNKS = E_PAD // (16 * CH)
ROWS_T = N_PAD // 16
B_ROWS = 1024
FC = 32


def _mm_body(h_ref, w_ref, o_ref):
    o_ref[...] = jnp.dot(h_ref[...], w_ref[...],
                         preferred_element_type=jnp.float32)


def _mm(h, w):
    n = h.shape[0]
    return pl.pallas_call(
        _mm_body,
        grid=(n // B_ROWS,),
        in_specs=[
            pl.BlockSpec((B_ROWS, h.shape[1]), lambda i: (i, 0)),
            pl.BlockSpec(w.shape, lambda i: (0, 0)),
        ],
        out_specs=pl.BlockSpec((B_ROWS, w.shape[1]), lambda i: (i, 0)),
        out_shape=jax.ShapeDtypeStruct((n, w.shape[1]), jnp.float32),
    )(h, w)


def _epi_body(agg_ref, deg_ref, h_ref, wl_ref, b_ref, wn_ref,
              hout_ref, yout_ref, *, relu):
    deg = deg_ref[...]
    nrm = 1.0 / jnp.maximum(deg, 1.0)
    agg = agg_ref[...]
    s = jnp.sum(agg * nrm, axis=0)
    h = s + jnp.dot(h_ref[...], wl_ref[...],
                    preferred_element_type=jnp.float32) + b_ref[...]
    if relu:
        h = jnp.maximum(h, 0.0)
    hout_ref[...] = h
    if yout_ref is not None:
        yout_ref[...] = jnp.dot(h, wn_ref[...],
                                preferred_element_type=jnp.float32)


def _epi(agg, deg, h, wl, b, wn, relu):
    d = agg.shape[2]
    have_y = wn is not None
    body = functools.partial(_epi_body, relu=relu)
    if not have_y:
        def body(agg_ref, deg_ref, h_ref, wl_ref, b_ref, hout_ref):
            _epi_body(agg_ref, deg_ref, h_ref, wl_ref, b_ref, None,
                      hout_ref, None, relu=relu)
    in_specs = [
        pl.BlockSpec((R, B_ROWS, d), lambda i: (0, i, 0)),
        pl.BlockSpec((R, B_ROWS, 1), lambda i: (0, i, 0)),
        pl.BlockSpec((B_ROWS, h.shape[1]), lambda i: (i, 0)),
        pl.BlockSpec(wl.shape, lambda i: (0, 0)),
        pl.BlockSpec(b.shape, lambda i: (0, 0)),
    ]
    out_shape = [jax.ShapeDtypeStruct((N_PAD, d), jnp.float32)]
    out_specs = [pl.BlockSpec((B_ROWS, d), lambda i: (i, 0))]
    args = [agg, deg, h, wl, b]
    if have_y:
        in_specs.append(pl.BlockSpec(wn.shape, lambda i: (0, 0)))
        out_shape.append(
            jax.ShapeDtypeStruct((N_PAD, wn.shape[1]), jnp.float32))
        out_specs.append(pl.BlockSpec((B_ROWS, wn.shape[1]), lambda i: (i, 0)))
        args.append(wn)
    res = pl.pallas_call(
        body,
        grid=(N_PAD // B_ROWS,),
        in_specs=in_specs,
        out_specs=out_specs,
        out_shape=out_shape,
    )(*args)
    return res if have_y else (res[0], None)


_MESH = plsc.VectorSubcoreMesh(core_axis_name="c", subcore_axis_name="s",
                               num_cores=2, num_subcores=16)
_SC_PARAMS = pltpu.CompilerParams(use_tc_tiling_on_sc=False)


def _fill(ref, n, value):
    def body(i, carry):
        ref[pl.ds(i * 16, 16)] = jnp.full((16,), value, jnp.float32)
        return carry
    lax.fori_loop(0, n // 16, body, 0)


def _deg_kernel(dst3_hbm, deg_hbm, ones_v, idx_v, zbuf, acc):
    c = lax.axis_index("c")
    t = lax.axis_index("s")
    _fill(ones_v, CH, 1.0)
    z = 3 * N_PAD // 16
    _fill(zbuf, z, 0.0)
    pltpu.sync_copy(zbuf, acc.at[pl.ds(t * z, z)])
    plsc.subcore_barrier()

    def run_rel(r):
        def body(i, carry):
            off = r * E_PAD + t * (CHUNKS * CH) + i * CH
            pltpu.sync_copy(dst3_hbm.at[pl.ds(off, CH)], idx_v)
            pltpu.sync_copy(ones_v, acc.at[idx_v], add=True)
            return carry
        lax.fori_loop(0, CHUNKS, body, 0)

    @pl.when(c == 0)
    def _():
        run_rel(0)
        run_rel(1)

    @pl.when(c == 1)
    def _():
        run_rel(2)

    plsc.subcore_barrier()

    @pl.when(c == 0)
    def _():
        sl = 2 * N_PAD // 16
        pltpu.sync_copy(acc.at[pl.ds(t * sl, sl)], zbuf.at[pl.ds(0, sl)])
        pltpu.sync_copy(zbuf.at[pl.ds(0, sl)], deg_hbm.at[pl.ds(t * sl, sl)])

    @pl.when(c == 1)
    def _():
        sl = N_PAD // 16
        pltpu.sync_copy(acc.at[pl.ds(2 * N_PAD + t * sl, sl)],
                        zbuf.at[pl.ds(0, sl)])
        pltpu.sync_copy(zbuf.at[pl.ds(0, sl)],
                        deg_hbm.at[pl.ds(2 * N_PAD + t * sl, sl)])


@functools.partial(
    pl.kernel,
    out_type=jax.ShapeDtypeStruct((3 * N_PAD,), jnp.float32),
    mesh=_MESH,
    scratch_types=[
        pltpu.VMEM((CH,), jnp.float32),
        pltpu.VMEM((CH,), jnp.int32),
        pltpu.VMEM((3 * N_PAD // 16,), jnp.float32),
        pltpu.VMEM_SHARED((3 * N_PAD,), jnp.float32),
    ],
    compiler_params=_SC_PARAMS,
)
def _deg(dst3_hbm, deg_hbm, ones_v, idx_v, zbuf, acc):
    _deg_kernel(dst3_hbm, deg_hbm, ones_v, idx_v, zbuf, acc)


def _fill2d(ref, nrows, value):
    def body(i, carry):
        for j in range(FC // 16):
            ref[i, pl.ds(j * 16, 16)] = jnp.full((16,), value, jnp.float32)
        return carry
    lax.fori_loop(0, nrows, body, 0)


def _make_agg(dout):
    nchunk = dout // FC
    passes = [(r, p) for r in range(R) for p in range(nchunk)]
    half = len(passes) // 2
    zrows = ROWS_T // 8
    nz = 8

    def body(y_hbm, src_hbm, dst_hbm, agg_hbm, acc, sem0, sem1):
        pl.run_scoped(
            functools.partial(_agg_scoped, y_hbm, src_hbm, dst_hbm,
                              agg_hbm, acc, sem0, sem1),
            sbuf0=pltpu.VMEM((CH,), jnp.int32),
            dbuf0=pltpu.VMEM((CH,), jnp.int32),
            sbuf1=pltpu.VMEM((CH,), jnp.int32),
            dbuf1=pltpu.VMEM((CH,), jnp.int32),
            rows0=pltpu.VMEM((CH, FC), jnp.float32),
            rows1=pltpu.VMEM((CH, FC), jnp.float32),
        )

    def _agg_scoped(y_hbm, src_hbm, dst_hbm, agg_hbm, acc, sem0, sem1,
                    sbuf0, dbuf0, sbuf1, dbuf1, rows0, rows1):
        c = lax.axis_index("c")
        t = lax.axis_index("s")
        half = (R * nchunk) // 2
        pairs = CHUNKS // 2

        def pass_body(q, carry):
            r = q // nchunk
            p = q - r * nchunk
            ebase = t * (CHUNKS * CH)

            def load_idx(i, sb, db):
                pltpu.sync_copy(src_hbm.at[pl.ds(q * E_PAD + ebase + i * CH,
                                                 CH)], sb)
                pltpu.sync_copy(dst_hbm.at[pl.ds(r * E_PAD + ebase + i * CH,
                                                 CH)], db)

            _fill2d(rows0, zrows, 0.0)

            def zero(zz, carry2):
                pltpu.sync_copy(rows0.at[pl.ds(0, zrows)],
                                acc.at[pl.ds(t * ROWS_T + zz * zrows, zrows)])
                return carry2
            lax.fori_loop(0, nz, zero, 0)
            plsc.subcore_barrier()

            load_idx(0, sbuf0, dbuf0)
            pltpu.async_copy(y_hbm.at[sbuf0], rows0, sem0)

            def pair(g, carry2):
                load_idx(2 * g + 1, sbuf1, dbuf1)
                pltpu.async_copy(y_hbm.at[sbuf1], rows1, sem1)
                pltpu.make_async_copy(y_hbm.at[sbuf0], rows0, sem0).wait()
                pltpu.sync_copy(rows0, acc.at[dbuf0], add=True)

                @pl.when(g < pairs - 1)
                def _():
                    load_idx(2 * g + 2, sbuf0, dbuf0)
                    pltpu.async_copy(y_hbm.at[sbuf0], rows0, sem0)

                pltpu.make_async_copy(y_hbm.at[sbuf1], rows1, sem1).wait()
                pltpu.sync_copy(rows1, acc.at[dbuf1], add=True)
                return carry2
            lax.fori_loop(0, pairs, pair, 0)
            plsc.subcore_barrier()

            def drain(zz, carry2):
                stage = rows0.at[pl.ds(0, zrows)]
                pltpu.sync_copy(
                    acc.at[pl.ds(t * ROWS_T + zz * zrows, zrows)], stage)
                pltpu.sync_copy(
                    stage,
                    agg_hbm.at[
                        pl.ds(r * N_PAD + t * ROWS_T + zz * zrows, zrows),
                        pl.ds(p * FC, FC)])
                return carry2
            lax.fori_loop(0, nz, drain, 0)
            return carry

        lax.fori_loop(c * half, (c + 1) * half, pass_body, 0)

    return pl.kernel(
        body,
        out_type=jax.ShapeDtypeStruct((R * N_PAD, dout), jnp.float32),
        mesh=_MESH,
        scratch_types=[
            pltpu.VMEM_SHARED((N_PAD, FC), jnp.float32),
            pltpu.SemaphoreType.DMA,
            pltpu.SemaphoreType.DMA,
        ],
        compiler_params=_SC_PARAMS,
    )


_agg128 = _make_agg(D_H)
_agg64 = _make_agg(D_OUT)


def kernel(x, edge_index_r0, edge_index_r1, edge_index_r2,
           W0_r0, W0_r1, W0_r2, W0_loop, b0_loop,
           W1_r0, W1_r1, W1_r2, W1_loop, b1_loop,
           W2_r0, W2_r1, W2_r2, W2_loop, b2_loop):
    pad_n = E_PAD - E
    pad_src = (jnp.arange(pad_n, dtype=jnp.int32) * 977) % N
    pad_dst = N + (jnp.arange(pad_n, dtype=jnp.int32) % (N_PAD - N))
    srcs, dsts, dst3s = [], [], []
    for r, ei in enumerate((edge_index_r0, edge_index_r1, edge_index_r2)):
        s = jnp.concatenate([ei[0], pad_src])
        d = jnp.concatenate([ei[1], pad_dst])
        srcs.append(s)
        dsts.append(d)
        dst3s.append(d + r * N_PAD)
    dst_flat = jnp.concatenate(dsts)
    dst3_flat = jnp.concatenate(dst3s)
    srcs_pad = jnp.stack(srcs)
    q12 = jnp.arange(12, dtype=jnp.int32).reshape(3, 4)
    src12 = (srcs_pad[:, None, :] * 12 + q12[:, :, None]).reshape(-1)
    q6 = jnp.arange(6, dtype=jnp.int32).reshape(3, 2)
    src6 = (srcs_pad[:, None, :] * 6 + q6[:, :, None]).reshape(-1)

    x_pad = jnp.pad(x, ((0, N_PAD - N), (0, 0)))
    w_all0 = jnp.concatenate([W0_r0, W0_r1, W0_r2], axis=1)
    w_all1 = jnp.concatenate([W1_r0, W1_r1, W1_r2], axis=1)
    w_all2 = jnp.concatenate([W2_r0, W2_r1, W2_r2], axis=1)
    b0 = b0_loop.reshape(1, -1)
    b1 = b1_loop.reshape(1, -1)
    b2 = b2_loop.reshape(1, -1)

    deg = _deg(dst3_flat).reshape(R, N_PAD, 1)

    y0 = _mm(x_pad, w_all0)
    agg0 = _agg128(y0.reshape(-1, FC), src12, dst_flat).reshape(R, N_PAD, D_H)
    h1, y1 = _epi(agg0, deg, x_pad, W0_loop, b0, w_all1, relu=True)

    agg1 = _agg128(y1.reshape(-1, FC), src12, dst_flat).reshape(R, N_PAD, D_H)
    h2, y2 = _epi(agg1, deg, h1, W1_loop, b1, w_all2, relu=True)

    agg2 = _agg64(y2.reshape(-1, FC), src6, dst_flat).reshape(R, N_PAD, D_OUT)
    h3, _ = _epi(agg2, deg, h2, W2_loop, b2, None, relu=False)

    return h3[:N]


# speedup vs baseline: 1.2390x; 1.0194x over previous
import functools

import jax
import jax.numpy as jnp
from jax import lax
from jax.experimental import pallas as pl
from jax.experimental.pallas import tpu as pltpu, tpu_sc as plsc

N = 50000
E = 200000
R = 3
D_IN, D_H, D_OUT = 128, 128, 64

N_PAD = 50176
E_PAD = 204800
CH = 400
CHUNKS = E_PAD // (16 * CH)
ROWS_T = N_PAD // 16
B_ROWS = 1024
FC = 32


def _mm_body(h_ref, w_ref, o_ref):
    o_ref[...] = jnp.dot(h_ref[...], w_ref[...],
                         preferred_element_type=jnp.float32)


def _mm(h, w):
    n = h.shape[0]
    return pl.pallas_call(
        _mm_body,
        grid=(n // B_ROWS,),
        in_specs=[
            pl.BlockSpec((B_ROWS, h.shape[1]), lambda i: (i, 0)),
            pl.BlockSpec(w.shape, lambda i: (0, 0)),
        ],
        out_specs=pl.BlockSpec((B_ROWS, w.shape[1]), lambda i: (i, 0)),
        out_shape=jax.ShapeDtypeStruct((n, w.shape[1]), jnp.float32),
    )(h, w)


def _epi_body(agg_ref, deg_ref, h_ref, wl_ref, b_ref, wn_ref,
              hout_ref, yout_ref, *, relu):
    deg = deg_ref[...]
    nrm = 1.0 / jnp.maximum(deg, 1.0)
    agg = agg_ref[...]
    s = jnp.sum(agg * nrm, axis=0)
    h = s + jnp.dot(h_ref[...], wl_ref[...],
                    preferred_element_type=jnp.float32) + b_ref[...]
    if relu:
        h = jnp.maximum(h, 0.0)
    hout_ref[...] = h
    if yout_ref is not None:
        yout_ref[...] = jnp.dot(h, wn_ref[...],
                                preferred_element_type=jnp.float32)


def _epi(agg, deg, h, wl, b, wn, relu):
    d = agg.shape[2]
    have_y = wn is not None
    body = functools.partial(_epi_body, relu=relu)
    if not have_y:
        def body(agg_ref, deg_ref, h_ref, wl_ref, b_ref, hout_ref):
            _epi_body(agg_ref, deg_ref, h_ref, wl_ref, b_ref, None,
                      hout_ref, None, relu=relu)
    in_specs = [
        pl.BlockSpec((R, B_ROWS, d), lambda i: (0, i, 0)),
        pl.BlockSpec((R, B_ROWS, 1), lambda i: (0, i, 0)),
        pl.BlockSpec((B_ROWS, h.shape[1]), lambda i: (i, 0)),
        pl.BlockSpec(wl.shape, lambda i: (0, 0)),
        pl.BlockSpec(b.shape, lambda i: (0, 0)),
    ]
    out_shape = [jax.ShapeDtypeStruct((N_PAD, d), jnp.float32)]
    out_specs = [pl.BlockSpec((B_ROWS, d), lambda i: (i, 0))]
    args = [agg, deg, h, wl, b]
    if have_y:
        in_specs.append(pl.BlockSpec(wn.shape, lambda i: (0, 0)))
        out_shape.append(
            jax.ShapeDtypeStruct((N_PAD, wn.shape[1]), jnp.float32))
        out_specs.append(pl.BlockSpec((B_ROWS, wn.shape[1]), lambda i: (i, 0)))
        args.append(wn)
    res = pl.pallas_call(
        body,
        grid=(N_PAD // B_ROWS,),
        in_specs=in_specs,
        out_specs=out_specs,
        out_shape=out_shape,
    )(*args)
    return res if have_y else (res[0], None)


_MESH = plsc.VectorSubcoreMesh(core_axis_name="c", subcore_axis_name="s",
                               num_cores=2, num_subcores=16)
_SC_PARAMS = pltpu.CompilerParams(use_tc_tiling_on_sc=False)


def _fill(ref, n, value):
    def body(i, carry):
        ref[pl.ds(i * 16, 16)] = jnp.full((16,), value, jnp.float32)
        return carry
    lax.fori_loop(0, n // 16, body, 0)


def _deg_kernel(dst3_hbm, deg_hbm, ones_v, idx_v, zbuf, acc):
    c = lax.axis_index("c")
    t = lax.axis_index("s")
    _fill(ones_v, CH, 1.0)
    z = 3 * N_PAD // 16
    _fill(zbuf, z, 0.0)
    pltpu.sync_copy(zbuf, acc.at[pl.ds(t * z, z)])
    plsc.subcore_barrier()

    def run_rel(r):
        def body(i, carry):
            off = r * E_PAD + t * (CHUNKS * CH) + i * CH
            pltpu.sync_copy(dst3_hbm.at[pl.ds(off, CH)], idx_v)
            pltpu.sync_copy(ones_v, acc.at[idx_v], add=True)
            return carry
        lax.fori_loop(0, CHUNKS, body, 0)

    @pl.when(c == 0)
    def _():
        run_rel(0)
        run_rel(1)

    @pl.when(c == 1)
    def _():
        run_rel(2)

    plsc.subcore_barrier()

    @pl.when(c == 0)
    def _():
        sl = 2 * N_PAD // 16
        pltpu.sync_copy(acc.at[pl.ds(t * sl, sl)], zbuf.at[pl.ds(0, sl)])
        pltpu.sync_copy(zbuf.at[pl.ds(0, sl)], deg_hbm.at[pl.ds(t * sl, sl)])

    @pl.when(c == 1)
    def _():
        sl = N_PAD // 16
        pltpu.sync_copy(acc.at[pl.ds(2 * N_PAD + t * sl, sl)],
                        zbuf.at[pl.ds(0, sl)])
        pltpu.sync_copy(zbuf.at[pl.ds(0, sl)],
                        deg_hbm.at[pl.ds(2 * N_PAD + t * sl, sl)])


@functools.partial(
    pl.kernel,
    out_type=jax.ShapeDtypeStruct((3 * N_PAD,), jnp.float32),
    mesh=_MESH,
    scratch_types=[
        pltpu.VMEM((CH,), jnp.float32),
        pltpu.VMEM((CH,), jnp.int32),
        pltpu.VMEM((3 * N_PAD // 16,), jnp.float32),
        pltpu.VMEM_SHARED((3 * N_PAD,), jnp.float32),
    ],
    compiler_params=_SC_PARAMS,
)
def _deg(dst3_hbm, deg_hbm, ones_v, idx_v, zbuf, acc):
    _deg_kernel(dst3_hbm, deg_hbm, ones_v, idx_v, zbuf, acc)


def _fill2d(ref, nrows, value):
    def body(i, carry):
        for j in range(FC // 16):
            ref[i, pl.ds(j * 16, 16)] = jnp.full((16,), value, jnp.float32)
        return carry
    lax.fori_loop(0, nrows, body, 0)


def _make_agg(dout):
    nchunk = dout // FC
    passes = [(r, p) for r in range(R) for p in range(nchunk)]
    half = len(passes) // 2
    zrows = ROWS_T // 8
    nz = 8

    def body(y_hbm, comb_hbm, agg_hbm, acc, sem0, sem1):
        pl.run_scoped(
            functools.partial(_agg_scoped, y_hbm, comb_hbm,
                              agg_hbm, acc, sem0, sem1),
            sd0=pltpu.VMEM((2, CH), jnp.int32),
            sd1=pltpu.VMEM((2, CH), jnp.int32),
            rows0=pltpu.VMEM((CH, FC), jnp.float32),
            rows1=pltpu.VMEM((CH, FC), jnp.float32),
        )

    def _agg_scoped(y_hbm, comb_hbm, agg_hbm, acc, sem0, sem1,
                    sd0, sd1, rows0, rows1):
        c = lax.axis_index("c")
        t = lax.axis_index("s")
        half = (R * nchunk) // 2
        pairs = CHUNKS // 2

        def pass_body(q, carry):
            r = q // nchunk
            p = q - r * nchunk
            rbase = (q * (E_PAD // CH) + t * CHUNKS) * 2

            def load_idx(i, sd):
                pltpu.sync_copy(comb_hbm.at[pl.ds(rbase + i * 2, 2)], sd)

            _fill2d(rows0, zrows, 0.0)

            def zero(zz, carry2):
                pltpu.sync_copy(rows0.at[pl.ds(0, zrows)],
                                acc.at[pl.ds(t * ROWS_T + zz * zrows, zrows)])
                return carry2
            lax.fori_loop(0, nz, zero, 0)
            plsc.subcore_barrier()

            load_idx(0, sd0)
            pltpu.async_copy(y_hbm.at[sd0.at[0]], rows0, sem0)

            def pair(g, carry2):
                load_idx(2 * g + 1, sd1)
                pltpu.async_copy(y_hbm.at[sd1.at[0]], rows1, sem1)
                pltpu.make_async_copy(y_hbm.at[sd0.at[0]], rows0,
                                      sem0).wait()
                pltpu.sync_copy(rows0, acc.at[sd0.at[1]], add=True)

                @pl.when(g < pairs - 1)
                def _():
                    load_idx(2 * g + 2, sd0)
                    pltpu.async_copy(y_hbm.at[sd0.at[0]], rows0, sem0)

                pltpu.make_async_copy(y_hbm.at[sd1.at[0]], rows1,
                                      sem1).wait()
                pltpu.sync_copy(rows1, acc.at[sd1.at[1]], add=True)
                return carry2
            lax.fori_loop(0, pairs, pair, 0)
            plsc.subcore_barrier()

            def drain(zz, carry2):
                stage = rows0.at[pl.ds(0, zrows)]
                pltpu.sync_copy(
                    acc.at[pl.ds(t * ROWS_T + zz * zrows, zrows)], stage)
                pltpu.sync_copy(
                    stage,
                    agg_hbm.at[
                        pl.ds(r * N_PAD + t * ROWS_T + zz * zrows, zrows),
                        pl.ds(p * FC, FC)])
                return carry2
            lax.fori_loop(0, nz, drain, 0)
            return carry

        lax.fori_loop(c * half, (c + 1) * half, pass_body, 0)

    return pl.kernel(
        body,
        out_type=jax.ShapeDtypeStruct((R * N_PAD, dout), jnp.float32),
        mesh=_MESH,
        scratch_types=[
            pltpu.VMEM_SHARED((N_PAD, FC), jnp.float32),
            pltpu.SemaphoreType.DMA,
            pltpu.SemaphoreType.DMA,
        ],
        compiler_params=_SC_PARAMS,
    )


_agg128 = _make_agg(D_H)
_agg64 = _make_agg(D_OUT)


def kernel(x, edge_index_r0, edge_index_r1, edge_index_r2,
           W0_r0, W0_r1, W0_r2, W0_loop, b0_loop,
           W1_r0, W1_r1, W1_r2, W1_loop, b1_loop,
           W2_r0, W2_r1, W2_r2, W2_loop, b2_loop):
    pad_n = E_PAD - E
    pad_src = (jnp.arange(pad_n, dtype=jnp.int32) * 977) % N
    pad_dst = N + (jnp.arange(pad_n, dtype=jnp.int32) % (N_PAD - N))
    srcs, dsts, dst3s = [], [], []
    for r, ei in enumerate((edge_index_r0, edge_index_r1, edge_index_r2)):
        s = jnp.concatenate([ei[0], pad_src])
        d = jnp.concatenate([ei[1], pad_dst])
        srcs.append(s)
        dsts.append(d)
        dst3s.append(d + r * N_PAD)
    dst_flat = jnp.concatenate(dsts)
    dst3_flat = jnp.concatenate(dst3s)
    srcs_pad = jnp.stack(srcs)
    q12 = jnp.arange(12, dtype=jnp.int32).reshape(3, 4)
    src12 = (srcs_pad[:, None, :] * 12 + q12[:, :, None]).reshape(-1)
    q6 = jnp.arange(6, dtype=jnp.int32).reshape(3, 2)
    src6 = (srcs_pad[:, None, :] * 6 + q6[:, :, None]).reshape(-1)

    x_pad = jnp.pad(x, ((0, N_PAD - N), (0, 0)))
    w_all0 = jnp.concatenate([W0_r0, W0_r1, W0_r2], axis=1)
    w_all1 = jnp.concatenate([W1_r0, W1_r1, W1_r2], axis=1)
    w_all2 = jnp.concatenate([W2_r0, W2_r1, W2_r2], axis=1)
    b0 = b0_loop.reshape(1, -1)
    b1 = b1_loop.reshape(1, -1)
    b2 = b2_loop.reshape(1, -1)

    deg = _deg(dst3_flat).reshape(R, N_PAD, 1)

    crows = E_PAD // CH
    src12_3d = src12.reshape(12, crows, CH)
    dst_3d = dst_flat.reshape(R, crows, CH)
    dstb12 = jnp.broadcast_to(dst_3d[:, None], (R, 4, crows, CH)
                              ).reshape(12, crows, CH)
    comb12 = jnp.stack([src12_3d, dstb12], axis=2).reshape(-1, CH)
    src6_3d = src6.reshape(6, crows, CH)
    dstb6 = jnp.broadcast_to(dst_3d[:, None], (R, 2, crows, CH)
                             ).reshape(6, crows, CH)
    comb6 = jnp.stack([src6_3d, dstb6], axis=2).reshape(-1, CH)

    y0 = _mm(x_pad, w_all0)
    agg0 = _agg128(y0.reshape(-1, FC), comb12).reshape(R, N_PAD, D_H)
    h1, y1 = _epi(agg0, deg, x_pad, W0_loop, b0, w_all1, relu=True)

    agg1 = _agg128(y1.reshape(-1, FC), comb12).reshape(R, N_PAD, D_H)
    h2, y2 = _epi(agg1, deg, h1, W1_loop, b1, w_all2, relu=True)

    agg2 = _agg64(y2.reshape(-1, FC), comb6).reshape(R, N_PAD, D_OUT)
    h3, _ = _epi(agg2, deg, h2, W2_loop, b2, None, relu=False)

    return h3[:N]
